# Initial kernel scaffold; baseline (speedup 1.0000x reference)
import jax
import jax.numpy as jnp
from jax.experimental import pallas as pl


def kernel(pred0, pred1, pred2):
    raise NotImplementedError("write your pallas kernel here")


# speedup vs baseline: 12.1951x; 12.1951x over previous
import numpy as np
import jax
import jax.numpy as jnp
from jax.experimental import pallas as pl

_INPUT_SIZE = 416.0
_ANCHORS = np.array([[[116, 90], [156, 198], [373, 326]],
                     [[30, 61], [62, 45], [59, 119]],
                     [[10, 13], [16, 30], [33, 23]]], dtype=np.float32)
_SCORE_THR = 0.5
_IOU_THR = 0.5
_MAX_DET = 100

_N = 16
_SCALES = (13, 26, 52)
_SEGS = []
_r0 = 0
for _S in _SCALES:
    _n = 3 * _S * _S
    _rows = -(-_n // 128)
    _SEGS.append((_r0, _rows, _n))
    _r0 += _rows
_ROWS_TOT = _r0
_NPAD = _ROWS_TOT * 128


def _make_consts():
    cx = np.zeros((_ROWS_TOT, 128), np.float32)
    cy = np.zeros((_ROWS_TOT, 128), np.float32)
    aw = np.ones((_ROWS_TOT, 128), np.float32)
    ah = np.ones((_ROWS_TOT, 128), np.float32)
    valid = np.zeros((_ROWS_TOT, 128), np.int32)
    for si, S in enumerate(_SCALES):
        r0, rows, n = _SEGS[si]
        idx = np.arange(rows * 128)
        a = np.minimum(idx // (S * S), 2)
        rem = idx % (S * S)
        cyv = (rem // S).astype(np.float32)
        cxv = (rem % S).astype(np.float32)
        blk = slice(r0, r0 + rows)
        cx[blk] = cxv.reshape(rows, 128)
        cy[blk] = cyv.reshape(rows, 128)
        aw[blk] = _ANCHORS[si][a, 0].reshape(rows, 128)
        ah[blk] = _ANCHORS[si][a, 1].reshape(rows, 128)
        v = (idx < n).astype(np.int32)
        valid[blk] = v.reshape(rows, 128)
    return cx, cy, aw, ah, valid


_CX, _CY, _AW, _AH, _VALID = _make_consts()


def _decode_body(p0, p1, p2, cx, cy, aw, ah, valid,
                 x1o, y1o, x2o, y2o, area, sthr, lab):
    preds = (p0, p1, p2)
    for si, S in enumerate(_SCALES):
        r0, rows, n = _SEGS[si]
        r = preds[si][0]
        lbl = jnp.argmax(r[:, 5:], axis=-1).astype(jnp.int32)

        pad = rows * 128 - n

        def to2d(v, fill):
            v = jnp.concatenate([v, jnp.full((pad,), fill, v.dtype)])
            return v.reshape(rows, 128)

        s2 = to2d(r[:, 0], 0.0)
        tx = to2d(r[:, 1], 0.0)
        ty = to2d(r[:, 2], 0.0)
        tw = to2d(r[:, 3], 0.0)
        th = to2d(r[:, 4], 0.0)
        lb = to2d(lbl, 0)

        sl = slice(r0, r0 + rows)
        cxs, cys = cx[sl], cy[sl]
        aws, ahs = aw[sl], ah[sl]
        vals = valid[sl]

        stride = _INPUT_SIZE / S
        score = jax.nn.sigmoid(s2)
        bx = (jax.nn.sigmoid(tx) + cxs) * stride
        by = (jax.nn.sigmoid(ty) + cys) * stride
        bw = aws * jnp.exp(tw)
        bh = ahs * jnp.exp(th)
        x1 = jnp.clip(bx - bw / 2.0, 0.0, _INPUT_SIZE)
        y1 = jnp.clip(by - bh / 2.0, 0.0, _INPUT_SIZE)
        x2 = jnp.clip(bx + bw / 2.0, 0.0, _INPUT_SIZE)
        y2 = jnp.clip(by + bh / 2.0, 0.0, _INPUT_SIZE)

        off = lb.astype(jnp.float32) * (_INPUT_SIZE + 1.0)
        x1v = x1 + off
        y1v = y1 + off
        x2v = x2 + off
        y2v = y2 + off
        av = (x2v - x1v) * (y2v - y1v)
        sv = jnp.where((vals != 0) & (score > _SCORE_THR), score,
                       -jnp.inf)

        x1o[0, sl, :] = x1v
        y1o[0, sl, :] = y1v
        x2o[0, sl, :] = x2v
        y2o[0, sl, :] = y2v
        area[0, sl, :] = av
        sthr[0, sl, :] = sv
        lab[0, sl, :] = lb


def _nms_body(x1o_r, y1o_r, x2o_r, y2o_r, area_r, sthr_r, lab_r,
              sout, lout, x1out, y1out, x2out, y2out):
    x1o = x1o_r[...]
    y1o = y1o_r[...]
    x2o = x2o_r[...]
    y2o = y2o_r[...]
    area = area_r[...]
    s0 = sthr_r[...]
    lab = lab_r[...]

    bidx = jax.lax.broadcasted_iota(jnp.int32, (_N, _NPAD), 1)
    lane = jax.lax.broadcasted_iota(jnp.int32, (_N, 128), 1)
    neg_inf = jnp.float32(-jnp.inf)
    big = jnp.int32(1 << 30)

    def body(i, carry):
        s, sacc, lacc, x1a, y1a, x2a, y2a = carry
        mval = jnp.max(s, axis=1, keepdims=True)
        valid = mval > neg_inf
        j = jnp.min(jnp.where(s == mval, bidx, big), axis=1, keepdims=True)
        mj = bidx == j
        x1j = jnp.sum(jnp.where(mj, x1o, 0.0), axis=1, keepdims=True)
        y1j = jnp.sum(jnp.where(mj, y1o, 0.0), axis=1, keepdims=True)
        x2j = jnp.sum(jnp.where(mj, x2o, 0.0), axis=1, keepdims=True)
        y2j = jnp.sum(jnp.where(mj, y2o, 0.0), axis=1, keepdims=True)
        lbj = jnp.sum(jnp.where(mj, lab, 0), axis=1, keepdims=True)
        areaj = (x2j - x1j) * (y2j - y1j)
        xx1 = jnp.maximum(x1j, x1o)
        yy1 = jnp.maximum(y1j, y1o)
        xx2 = jnp.minimum(x2j, x2o)
        yy2 = jnp.minimum(y2j, y2o)
        inter = jnp.clip(xx2 - xx1, 0.0) * jnp.clip(yy2 - yy1, 0.0)
        iou = inter / (areaj + area - inter + 1e-9)
        s_new = jnp.where((iou > _IOU_THR) | mj, neg_inf, s)
        s = jnp.where(valid, s_new, s)

        lm = lane == i
        keep = lm & valid
        offj = lbj.astype(jnp.float32) * (_INPUT_SIZE + 1.0)
        sacc = jnp.where(keep, mval, sacc)
        lacc = jnp.where(lm, jnp.where(valid, lbj, -1), lacc)
        x1a = jnp.where(keep, x1j - offj, x1a)
        y1a = jnp.where(keep, y1j - offj, y1a)
        x2a = jnp.where(keep, x2j - offj, x2a)
        y2a = jnp.where(keep, y2j - offj, y2a)
        return s, sacc, lacc, x1a, y1a, x2a, y2a

    zf = jnp.zeros((_N, 128), jnp.float32)
    init = (s0, zf, jnp.full((_N, 128), -1, jnp.int32), zf, zf, zf, zf)
    _, sacc, lacc, x1a, y1a, x2a, y2a = jax.lax.fori_loop(
        0, _MAX_DET, body, init)

    sout[...] = sacc
    lout[...] = lacc
    x1out[...] = x1a
    y1out[...] = y1a
    x2out[...] = x2a
    y2out[...] = y2a


def kernel(pred0, pred1, pred2):
    p0 = pred0.reshape(_N, 3 * 13 * 13, 85)
    p1 = pred1.reshape(_N, 3 * 26 * 26, 85)
    p2 = pred2.reshape(_N, 3 * 52 * 52, 85)

    consts = (jnp.asarray(_CX), jnp.asarray(_CY), jnp.asarray(_AW),
              jnp.asarray(_AH), jnp.asarray(_VALID))

    f32 = jnp.float32
    dec_out = [jax.ShapeDtypeStruct((_N, _ROWS_TOT, 128), f32)] * 6
    dec_out.append(jax.ShapeDtypeStruct((_N, _ROWS_TOT, 128), jnp.int32))

    in_specs = [
        pl.BlockSpec((1, p0.shape[1], 85), lambda b: (b, 0, 0)),
        pl.BlockSpec((1, p1.shape[1], 85), lambda b: (b, 0, 0)),
        pl.BlockSpec((1, p2.shape[1], 85), lambda b: (b, 0, 0)),
    ] + [pl.BlockSpec((_ROWS_TOT, 128), lambda b: (0, 0))] * 5
    out_specs = [pl.BlockSpec((1, _ROWS_TOT, 128), lambda b: (b, 0, 0))] * 7

    x1o, y1o, x2o, y2o, area, sthr, lab = pl.pallas_call(
        _decode_body,
        grid=(_N,),
        in_specs=in_specs,
        out_specs=out_specs,
        out_shape=dec_out,
    )(p0, p1, p2, *consts)

    flat = lambda a: a.reshape(_N, _NPAD)
    nms_out = ([jax.ShapeDtypeStruct((_N, 128), f32),
                jax.ShapeDtypeStruct((_N, 128), jnp.int32)]
               + [jax.ShapeDtypeStruct((_N, 128), f32)] * 4)

    sacc, lacc, x1a, y1a, x2a, y2a = pl.pallas_call(
        _nms_body,
        out_shape=nms_out,
    )(flat(x1o), flat(y1o), flat(x2o), flat(y2o), flat(area),
      flat(sthr), flat(lab))

    b_out = jnp.stack([x1a, y1a, x2a, y2a], axis=-1)[:, :_MAX_DET, :]
    s_out = sacc[:, :_MAX_DET]
    l_out = lacc[:, :_MAX_DET]
    return b_out, s_out, l_out


# speedup vs baseline: 16.5126x; 1.3540x over previous
import functools

import numpy as np
import jax
import jax.numpy as jnp
from jax import lax
from jax.experimental import pallas as pl
from jax.experimental.pallas import tpu as pltpu
from jax.experimental.pallas import tpu_sc as plsc

_INPUT_SIZE = 416.0
_ANCHORS = np.array([[[116, 90], [156, 198], [373, 326]],
                     [[30, 61], [62, 45], [59, 119]],
                     [[10, 13], [16, 30], [33, 23]]], dtype=np.float32)
_SCORE_THR = 0.5
_IOU_THR = 0.5
_MAX_DET = 100

_N = 16
_SCALES = (13, 26, 52)
_SEGS = []
_r0 = 0
for _S in _SCALES:
    _n = 3 * _S * _S
    _rows = -(-_n // 128)
    _SEGS.append((_r0, _rows, _n))
    _r0 += _rows
_ROWS_TOT = _r0
_NPAD = _ROWS_TOT * 128


def _make_consts():
    cx = np.zeros((_ROWS_TOT, 128), np.float32)
    cy = np.zeros((_ROWS_TOT, 128), np.float32)
    aw = np.ones((_ROWS_TOT, 128), np.float32)
    ah = np.ones((_ROWS_TOT, 128), np.float32)
    valid = np.zeros((_ROWS_TOT, 128), np.int32)
    for si, S in enumerate(_SCALES):
        r0, rows, n = _SEGS[si]
        idx = np.arange(rows * 128)
        a = np.minimum(idx // (S * S), 2)
        rem = idx % (S * S)
        cyv = (rem // S).astype(np.float32)
        cxv = (rem % S).astype(np.float32)
        blk = slice(r0, r0 + rows)
        cx[blk] = cxv.reshape(rows, 128)
        cy[blk] = cyv.reshape(rows, 128)
        aw[blk] = _ANCHORS[si][a, 0].reshape(rows, 128)
        ah[blk] = _ANCHORS[si][a, 1].reshape(rows, 128)
        v = (idx < n).astype(np.int32)
        valid[blk] = v.reshape(rows, 128)
    return cx, cy, aw, ah, valid


_CX, _CY, _AW, _AH, _VALID = _make_consts()


def _decode_body(p0, p1, p2, cx, cy, aw, ah, valid,
                 x1o, y1o, x2o, y2o, area, sthr, lab):
    preds = (p0, p1, p2)
    for si, S in enumerate(_SCALES):
        r0, rows, n = _SEGS[si]
        r = preds[si][0]
        lbl = jnp.argmax(r[:, 5:], axis=-1).astype(jnp.int32)

        pad = rows * 128 - n

        def to2d(v, fill):
            v = jnp.concatenate([v, jnp.full((pad,), fill, v.dtype)])
            return v.reshape(rows, 128)

        s2 = to2d(r[:, 0], 0.0)
        tx = to2d(r[:, 1], 0.0)
        ty = to2d(r[:, 2], 0.0)
        tw = to2d(r[:, 3], 0.0)
        th = to2d(r[:, 4], 0.0)
        lb = to2d(lbl, 0)

        sl = slice(r0, r0 + rows)
        cxs, cys = cx[sl], cy[sl]
        aws, ahs = aw[sl], ah[sl]
        vals = valid[sl]

        stride = _INPUT_SIZE / S
        score = jax.nn.sigmoid(s2)
        bx = (jax.nn.sigmoid(tx) + cxs) * stride
        by = (jax.nn.sigmoid(ty) + cys) * stride
        bw = aws * jnp.exp(tw)
        bh = ahs * jnp.exp(th)
        x1 = jnp.clip(bx - bw / 2.0, 0.0, _INPUT_SIZE)
        y1 = jnp.clip(by - bh / 2.0, 0.0, _INPUT_SIZE)
        x2 = jnp.clip(bx + bw / 2.0, 0.0, _INPUT_SIZE)
        y2 = jnp.clip(by + bh / 2.0, 0.0, _INPUT_SIZE)

        off = lb.astype(jnp.float32) * (_INPUT_SIZE + 1.0)
        x1v = x1 + off
        y1v = y1 + off
        x2v = x2 + off
        y2v = y2 + off
        av = (x2v - x1v) * (y2v - y1v)
        sv = jnp.where((vals != 0) & (score > _SCORE_THR), score,
                       -jnp.inf)

        x1o[0, sl, :] = x1v
        y1o[0, sl, :] = y1v
        x2o[0, sl, :] = x2v
        y2o[0, sl, :] = y2v
        area[0, sl, :] = av
        sthr[0, sl, :] = sv
        lab[0, sl, :] = lb


_NC, _NS = 2, 16
_NVREG = _NPAD // 16
_NCACHE1 = -(-_NVREG // 16)
_L2PAD = 48


def _sc_nms_body(x1h, y1h, x2h, y2h, sh, labh,
                 south, louth, ox1h, oy1h, ox2h, oy2h,
                 x1v, y1v, x2v, y2v, sv, labv,
                 cachev, l2v, kx1, ky1, kx2, ky2, karea,
                 outs, outl, o1, o2, o3, o4):
    wid = lax.axis_index("s") * _NC + lax.axis_index("c")

    @pl.when(wid < _N)
    def _():
        b = wid
        pltpu.sync_copy(x1h.at[b], x1v)
        pltpu.sync_copy(y1h.at[b], y1v)
        pltpu.sync_copy(x2h.at[b], x2v)
        pltpu.sync_copy(y2h.at[b], y2v)
        pltpu.sync_copy(sh.at[b], sv)
        pltpu.sync_copy(labh.at[b], labv)

        neg = jnp.float32(-jnp.inf)
        f32 = jnp.float32
        iota = lax.iota(jnp.int32, 16)
        big_i = jnp.int32(1 << 30)
        sent16 = jnp.full((16,), 1e9, f32)
        zero16 = jnp.zeros((16,), f32)
        neg16 = jnp.full((16,), -jnp.inf, f32)
        lane0 = iota == 0

        def vld1(ref, i):
            return plsc.load_gather(ref, [jnp.full((16,), i, jnp.int32)])

        def vst1(ref, i, val16):
            plsc.store_scatter(ref, [jnp.full((16,), i, jnp.int32)],
                               val16, mask=lane0)

        for t in range(7):
            kx1[pl.ds(t * 16, 16)] = sent16
            ky1[pl.ds(t * 16, 16)] = sent16
            kx2[pl.ds(t * 16, 16)] = sent16
            ky2[pl.ds(t * 16, 16)] = sent16
            karea[pl.ds(t * 16, 16)] = zero16
        for t in range(8):
            outs[pl.ds(t * 16, 16)] = zero16
            outl[pl.ds(t * 16, 16)] = jnp.full((16,), -1, jnp.int32)
            o1[pl.ds(t * 16, 16)] = zero16
            o2[pl.ds(t * 16, 16)] = zero16
            o3[pl.ds(t * 16, 16)] = zero16
            o4[pl.ds(t * 16, 16)] = zero16

        for t in range(_NCACHE1, 48):
            cachev[pl.ds(t * 16, 16)] = neg16

        def cb(ci, _):
            m = jnp.max(sv[pl.ds(pl.multiple_of(ci * 16, 16), 16)])
            vst1(cachev, ci, jnp.broadcast_to(m, (16,)))
            return 0
        lax.fori_loop(0, _NVREG, cb, 0, unroll=8)

        for t in range(3):
            l2v[pl.ds(t * 16, 16)] = neg16

        def l2b(q, _):
            m = jnp.max(cachev[pl.ds(pl.multiple_of(q * 16, 16), 16)])
            vst1(l2v, q, jnp.broadcast_to(m, (16,)))
            return 0
        lax.fori_loop(0, _NCACHE1, l2b, 0, unroll=8)

        def locate():
            m = jnp.maximum(l2v[pl.ds(0, 16)], l2v[pl.ds(16, 16)])
            m = jnp.maximum(m, l2v[pl.ds(32, 16)])
            gmax = jnp.max(m)
            qidx = big_i
            for t in range(3):
                v = l2v[pl.ds(t * 16, 16)]
                qidx = jnp.minimum(
                    qidx, jnp.min(jnp.where(v == gmax, iota + t * 16, big_i)))
            q = jnp.minimum(qidx, _NCACHE1 - 1)
            cvec = cachev[pl.ds(pl.multiple_of(q * 16, 16), 16)]
            lq = jnp.min(jnp.where(cvec == gmax, iota, big_i))
            ci = jnp.minimum(q * 16 + lq, _NVREG - 1)
            svec = sv[pl.ds(pl.multiple_of(ci * 16, 16), 16)]
            ll = jnp.minimum(
                jnp.min(jnp.where(svec == gmax, iota, big_i)), 15)
            return gmax, ci * 16 + ll

        def cond(carry):
            nk, gmax, j = carry
            return (nk < _MAX_DET) & (gmax > neg)

        def body(carry):
            nk, gmax, j = carry
            x1c = vld1(x1v, j)
            y1c = vld1(y1v, j)
            x2c = vld1(x2v, j)
            y2c = vld1(y2v, j)
            labc = vld1(labv, j)
            areac = (x2c - x1c) * (y2c - y1c)

            miou = jnp.zeros((16,), f32)
            for t in range(7):
                ka1 = kx1[pl.ds(t * 16, 16)]
                kb1 = ky1[pl.ds(t * 16, 16)]
                ka2 = kx2[pl.ds(t * 16, 16)]
                kb2 = ky2[pl.ds(t * 16, 16)]
                kar = karea[pl.ds(t * 16, 16)]
                xx1 = jnp.maximum(ka1, x1c)
                yy1 = jnp.maximum(kb1, y1c)
                xx2 = jnp.minimum(ka2, x2c)
                yy2 = jnp.minimum(kb2, y2c)
                inter = (jnp.clip(xx2 - xx1, 0.0) *
                         jnp.clip(yy2 - yy1, 0.0))
                iou = inter / (kar + areac - inter + 1e-9)
                miou = jnp.maximum(miou, iou)
            supp = jnp.max(miou) > _IOU_THR

            @pl.when(jnp.logical_not(supp))
            def _():
                vst1(kx1, nk, x1c)
                vst1(ky1, nk, y1c)
                vst1(kx2, nk, x2c)
                vst1(ky2, nk, y2c)
                vst1(karea, nk, areac)
                off = labc.astype(f32) * (_INPUT_SIZE + 1.0)
                vst1(outs, nk, jnp.broadcast_to(gmax, (16,)))
                vst1(outl, nk, labc)
                vst1(o1, nk, x1c - off)
                vst1(o2, nk, y1c - off)
                vst1(o3, nk, x2c - off)
                vst1(o4, nk, y2c - off)

            nk = nk + jnp.where(supp, jnp.int32(0), jnp.int32(1))

            vst1(sv, j, neg16)
            ci = j // 16
            q = ci // 16
            m1 = jnp.max(sv[pl.ds(pl.multiple_of(ci * 16, 16), 16)])
            vst1(cachev, ci, jnp.broadcast_to(m1, (16,)))
            m2 = jnp.max(cachev[pl.ds(pl.multiple_of(q * 16, 16), 16)])
            vst1(l2v, q, jnp.broadcast_to(m2, (16,)))
            gmax2, j2 = locate()
            return nk, gmax2, j2

        g0, j0 = locate()
        lax.while_loop(cond, body, (jnp.int32(0), g0, j0))

        pltpu.sync_copy(outs, south.at[b])
        pltpu.sync_copy(outl, louth.at[b])
        pltpu.sync_copy(o1, ox1h.at[b])
        pltpu.sync_copy(o2, oy1h.at[b])
        pltpu.sync_copy(o3, ox2h.at[b])
        pltpu.sync_copy(o4, oy2h.at[b])


def _nms_body(x1o_r, y1o_r, x2o_r, y2o_r, area_r, sthr_r, lab_r,
              sout, lout, x1out, y1out, x2out, y2out):
    x1o = x1o_r[...]
    y1o = y1o_r[...]
    x2o = x2o_r[...]
    y2o = y2o_r[...]
    area = area_r[...]
    s0 = sthr_r[...]
    lab = lab_r[...]

    bidx = jax.lax.broadcasted_iota(jnp.int32, (_N, _NPAD), 1)
    lane = jax.lax.broadcasted_iota(jnp.int32, (_N, 128), 1)
    neg_inf = jnp.float32(-jnp.inf)
    big = jnp.int32(1 << 30)

    def body(i, carry):
        s, sacc, lacc, x1a, y1a, x2a, y2a = carry
        mval = jnp.max(s, axis=1, keepdims=True)
        valid = mval > neg_inf
        j = jnp.min(jnp.where(s == mval, bidx, big), axis=1, keepdims=True)
        mj = bidx == j
        x1j = jnp.sum(jnp.where(mj, x1o, 0.0), axis=1, keepdims=True)
        y1j = jnp.sum(jnp.where(mj, y1o, 0.0), axis=1, keepdims=True)
        x2j = jnp.sum(jnp.where(mj, x2o, 0.0), axis=1, keepdims=True)
        y2j = jnp.sum(jnp.where(mj, y2o, 0.0), axis=1, keepdims=True)
        lbj = jnp.sum(jnp.where(mj, lab, 0), axis=1, keepdims=True)
        areaj = (x2j - x1j) * (y2j - y1j)
        xx1 = jnp.maximum(x1j, x1o)
        yy1 = jnp.maximum(y1j, y1o)
        xx2 = jnp.minimum(x2j, x2o)
        yy2 = jnp.minimum(y2j, y2o)
        inter = jnp.clip(xx2 - xx1, 0.0) * jnp.clip(yy2 - yy1, 0.0)
        iou = inter / (areaj + area - inter + 1e-9)
        s_new = jnp.where((iou > _IOU_THR) | mj, neg_inf, s)
        s = jnp.where(valid, s_new, s)

        lm = lane == i
        keep = lm & valid
        offj = lbj.astype(jnp.float32) * (_INPUT_SIZE + 1.0)
        sacc = jnp.where(keep, mval, sacc)
        lacc = jnp.where(lm, jnp.where(valid, lbj, -1), lacc)
        x1a = jnp.where(keep, x1j - offj, x1a)
        y1a = jnp.where(keep, y1j - offj, y1a)
        x2a = jnp.where(keep, x2j - offj, x2a)
        y2a = jnp.where(keep, y2j - offj, y2a)
        return s, sacc, lacc, x1a, y1a, x2a, y2a

    zf = jnp.zeros((_N, 128), jnp.float32)
    init = (s0, zf, jnp.full((_N, 128), -1, jnp.int32), zf, zf, zf, zf)
    _, sacc, lacc, x1a, y1a, x2a, y2a = jax.lax.fori_loop(
        0, _MAX_DET, body, init)

    sout[...] = sacc
    lout[...] = lacc
    x1out[...] = x1a
    y1out[...] = y1a
    x2out[...] = x2a
    y2out[...] = y2a


def kernel(pred0, pred1, pred2):
    p0 = pred0.reshape(_N, 3 * 13 * 13, 85)
    p1 = pred1.reshape(_N, 3 * 26 * 26, 85)
    p2 = pred2.reshape(_N, 3 * 52 * 52, 85)

    consts = (jnp.asarray(_CX), jnp.asarray(_CY), jnp.asarray(_AW),
              jnp.asarray(_AH), jnp.asarray(_VALID))

    f32 = jnp.float32
    dec_out = [jax.ShapeDtypeStruct((_N, _ROWS_TOT, 128), f32)] * 6
    dec_out.append(jax.ShapeDtypeStruct((_N, _ROWS_TOT, 128), jnp.int32))

    in_specs = [
        pl.BlockSpec((1, p0.shape[1], 85), lambda b: (b, 0, 0)),
        pl.BlockSpec((1, p1.shape[1], 85), lambda b: (b, 0, 0)),
        pl.BlockSpec((1, p2.shape[1], 85), lambda b: (b, 0, 0)),
    ] + [pl.BlockSpec((_ROWS_TOT, 128), lambda b: (0, 0))] * 5
    out_specs = [pl.BlockSpec((1, _ROWS_TOT, 128), lambda b: (b, 0, 0))] * 7

    x1o, y1o, x2o, y2o, area, sthr, lab = pl.pallas_call(
        _decode_body,
        grid=(_N,),
        in_specs=in_specs,
        out_specs=out_specs,
        out_shape=dec_out,
    )(p0, p1, p2, *consts)

    flat = lambda a: a.reshape(_N, _NPAD)

    sc_nms = functools.partial(
        pl.kernel,
        out_type=[jax.ShapeDtypeStruct((_N, 128), f32),
                  jax.ShapeDtypeStruct((_N, 128), jnp.int32)]
        + [jax.ShapeDtypeStruct((_N, 128), f32)] * 4,
        mesh=plsc.VectorSubcoreMesh(core_axis_name="c", subcore_axis_name="s",
                                    num_cores=_NC, num_subcores=_NS),
        compiler_params=pltpu.CompilerParams(needs_layout_passes=False),
        scratch_types=[pltpu.VMEM((_NPAD,), f32)] * 5
        + [pltpu.VMEM((_NPAD,), jnp.int32),
           pltpu.VMEM((768,), f32),
           pltpu.VMEM((_L2PAD,), f32)]
        + [pltpu.VMEM((112,), f32)] * 5
        + [pltpu.VMEM((128,), f32),
           pltpu.VMEM((128,), jnp.int32)]
        + [pltpu.VMEM((128,), f32)] * 4,
    )(_sc_nms_body)

    sacc, lacc, x1a, y1a, x2a, y2a = sc_nms(
        flat(x1o), flat(y1o), flat(x2o), flat(y2o), flat(sthr), flat(lab))

    b_out = jnp.stack([x1a, y1a, x2a, y2a], axis=-1)[:, :_MAX_DET, :]
    s_out = sacc[:, :_MAX_DET]
    l_out = lacc[:, :_MAX_DET]
    return b_out, s_out, l_out


# speedup vs baseline: 16.7898x; 1.0168x over previous
import functools

import numpy as np
import jax
import jax.numpy as jnp
from jax import lax
from jax.experimental import pallas as pl
from jax.experimental.pallas import tpu as pltpu
from jax.experimental.pallas import tpu_sc as plsc

_INPUT_SIZE = 416.0
_ANCHORS = np.array([[[116, 90], [156, 198], [373, 326]],
                     [[30, 61], [62, 45], [59, 119]],
                     [[10, 13], [16, 30], [33, 23]]], dtype=np.float32)
_SCORE_THR = 0.5
_IOU_THR = 0.5
_MAX_DET = 100

_N = 16
_SCALES = (13, 26, 52)
_SEGS = []
_r0 = 0
for _S in _SCALES:
    _n = 3 * _S * _S
    _rows = -(-_n // 128)
    _SEGS.append((_r0, _rows, _n))
    _r0 += _rows
_ROWS_TOT = _r0
_NPAD = _ROWS_TOT * 128


def _make_consts():
    cx = np.zeros((_ROWS_TOT, 128), np.float32)
    cy = np.zeros((_ROWS_TOT, 128), np.float32)
    aw = np.ones((_ROWS_TOT, 128), np.float32)
    ah = np.ones((_ROWS_TOT, 128), np.float32)
    valid = np.zeros((_ROWS_TOT, 128), np.int32)
    for si, S in enumerate(_SCALES):
        r0, rows, n = _SEGS[si]
        idx = np.arange(rows * 128)
        a = np.minimum(idx // (S * S), 2)
        rem = idx % (S * S)
        cyv = (rem // S).astype(np.float32)
        cxv = (rem % S).astype(np.float32)
        blk = slice(r0, r0 + rows)
        cx[blk] = cxv.reshape(rows, 128)
        cy[blk] = cyv.reshape(rows, 128)
        aw[blk] = _ANCHORS[si][a, 0].reshape(rows, 128)
        ah[blk] = _ANCHORS[si][a, 1].reshape(rows, 128)
        v = (idx < n).astype(np.int32)
        valid[blk] = v.reshape(rows, 128)
    return cx, cy, aw, ah, valid


_CX, _CY, _AW, _AH, _VALID = _make_consts()


def _decode_body(p0, p1, p2, q0, q1, q2, cx, cy, aw, ah, valid,
                 x1o, y1o, x2o, y2o, sthr, lab):
    preds = (p0, p1, p2)
    planes = (q0, q1, q2)
    for si, S in enumerate(_SCALES):
        r0, rows, n = _SEGS[si]
        r = preds[si][0]
        lbl = jnp.argmax(r[:, 5:], axis=-1).astype(jnp.int32)

        pad = rows * 128 - n

        def to2d(v, fill):
            v = jnp.concatenate([v, jnp.full((pad,), fill, v.dtype)])
            return v.reshape(rows, 128)

        q = planes[si][0]
        s2 = q[0 * rows:1 * rows, :]
        tx = q[1 * rows:2 * rows, :]
        ty = q[2 * rows:3 * rows, :]
        tw = q[3 * rows:4 * rows, :]
        th = q[4 * rows:5 * rows, :]
        lb = to2d(lbl, 0)

        sl = slice(r0, r0 + rows)
        cxs, cys = cx[sl], cy[sl]
        aws, ahs = aw[sl], ah[sl]
        vals = valid[sl]

        stride = _INPUT_SIZE / S
        score = jax.nn.sigmoid(s2)
        bx = (jax.nn.sigmoid(tx) + cxs) * stride
        by = (jax.nn.sigmoid(ty) + cys) * stride
        bw = aws * jnp.exp(tw)
        bh = ahs * jnp.exp(th)
        x1 = jnp.clip(bx - bw / 2.0, 0.0, _INPUT_SIZE)
        y1 = jnp.clip(by - bh / 2.0, 0.0, _INPUT_SIZE)
        x2 = jnp.clip(bx + bw / 2.0, 0.0, _INPUT_SIZE)
        y2 = jnp.clip(by + bh / 2.0, 0.0, _INPUT_SIZE)

        off = lb.astype(jnp.float32) * (_INPUT_SIZE + 1.0)
        x1v = x1 + off
        y1v = y1 + off
        x2v = x2 + off
        y2v = y2 + off
        sv = jnp.where((vals != 0) & (score > _SCORE_THR), score,
                       -jnp.inf)

        x1o[0, sl, :] = x1v
        y1o[0, sl, :] = y1v
        x2o[0, sl, :] = x2v
        y2o[0, sl, :] = y2v
        sthr[0, sl, :] = sv
        lab[0, sl, :] = lb


_NC, _NS = 2, 16
_NVREG = _NPAD // 16
_NCACHE1 = -(-_NVREG // 16)
_L2PAD = 48


def _sc_nms_body(x1h, y1h, x2h, y2h, sh, labh,
                 south, louth, ox1h, oy1h, ox2h, oy2h,
                 x1v, y1v, x2v, y2v, sv, labv,
                 cachev, l2v, kx1, ky1, kx2, ky2, karea,
                 outs, outl, o1, o2, o3, o4):
    wid = lax.axis_index("s") * _NC + lax.axis_index("c")

    @pl.when(wid < _N)
    def _():
        b = wid
        pltpu.sync_copy(x1h.at[b], x1v)
        pltpu.sync_copy(y1h.at[b], y1v)
        pltpu.sync_copy(x2h.at[b], x2v)
        pltpu.sync_copy(y2h.at[b], y2v)
        pltpu.sync_copy(sh.at[b], sv)
        pltpu.sync_copy(labh.at[b], labv)

        neg = jnp.float32(-jnp.inf)
        f32 = jnp.float32
        iota = lax.iota(jnp.int32, 16)
        big_i = jnp.int32(1 << 30)
        sent16 = jnp.full((16,), 1e9, f32)
        zero16 = jnp.zeros((16,), f32)
        neg16 = jnp.full((16,), -jnp.inf, f32)
        lane0 = iota == 0

        def vld1(ref, i):
            return plsc.load_gather(ref, [jnp.full((16,), i, jnp.int32)])

        def vst1(ref, i, val16):
            plsc.store_scatter(ref, [jnp.full((16,), i, jnp.int32)],
                               val16, mask=lane0)

        for t in range(7):
            kx1[pl.ds(t * 16, 16)] = sent16
            ky1[pl.ds(t * 16, 16)] = sent16
            kx2[pl.ds(t * 16, 16)] = sent16
            ky2[pl.ds(t * 16, 16)] = sent16
            karea[pl.ds(t * 16, 16)] = zero16
        for t in range(8):
            outs[pl.ds(t * 16, 16)] = zero16
            outl[pl.ds(t * 16, 16)] = jnp.full((16,), -1, jnp.int32)
            o1[pl.ds(t * 16, 16)] = zero16
            o2[pl.ds(t * 16, 16)] = zero16
            o3[pl.ds(t * 16, 16)] = zero16
            o4[pl.ds(t * 16, 16)] = zero16

        for t in range(_NCACHE1, 48):
            cachev[pl.ds(t * 16, 16)] = neg16

        def cb(ci, _):
            m = jnp.max(sv[pl.ds(pl.multiple_of(ci * 16, 16), 16)])
            vst1(cachev, ci, jnp.broadcast_to(m, (16,)))
            return 0
        lax.fori_loop(0, _NVREG, cb, 0, unroll=8)

        for t in range(3):
            l2v[pl.ds(t * 16, 16)] = neg16

        def l2b(q, _):
            m = jnp.max(cachev[pl.ds(pl.multiple_of(q * 16, 16), 16)])
            vst1(l2v, q, jnp.broadcast_to(m, (16,)))
            return 0
        lax.fori_loop(0, _NCACHE1, l2b, 0, unroll=8)

        def locate():
            m = jnp.maximum(l2v[pl.ds(0, 16)], l2v[pl.ds(16, 16)])
            m = jnp.maximum(m, l2v[pl.ds(32, 16)])
            gmax = jnp.max(m)
            qidx = big_i
            for t in range(3):
                v = l2v[pl.ds(t * 16, 16)]
                qidx = jnp.minimum(
                    qidx, jnp.min(jnp.where(v == gmax, iota + t * 16, big_i)))
            q = jnp.minimum(qidx, _NCACHE1 - 1)
            cvec = cachev[pl.ds(pl.multiple_of(q * 16, 16), 16)]
            lq = jnp.min(jnp.where(cvec == gmax, iota, big_i))
            ci = jnp.minimum(q * 16 + lq, _NVREG - 1)
            svec = sv[pl.ds(pl.multiple_of(ci * 16, 16), 16)]
            ll = jnp.minimum(
                jnp.min(jnp.where(svec == gmax, iota, big_i)), 15)
            return gmax, ci * 16 + ll

        def cond(carry):
            nk, gmax, j = carry
            return (nk < _MAX_DET) & (gmax > neg)

        def body(carry):
            nk, gmax, j = carry
            x1c = vld1(x1v, j)
            y1c = vld1(y1v, j)
            x2c = vld1(x2v, j)
            y2c = vld1(y2v, j)
            labc = vld1(labv, j)
            areac = (x2c - x1c) * (y2c - y1c)

            miou = jnp.zeros((16,), f32)
            for t in range(7):
                ka1 = kx1[pl.ds(t * 16, 16)]
                kb1 = ky1[pl.ds(t * 16, 16)]
                ka2 = kx2[pl.ds(t * 16, 16)]
                kb2 = ky2[pl.ds(t * 16, 16)]
                kar = karea[pl.ds(t * 16, 16)]
                xx1 = jnp.maximum(ka1, x1c)
                yy1 = jnp.maximum(kb1, y1c)
                xx2 = jnp.minimum(ka2, x2c)
                yy2 = jnp.minimum(kb2, y2c)
                inter = (jnp.clip(xx2 - xx1, 0.0) *
                         jnp.clip(yy2 - yy1, 0.0))
                iou = inter / (kar + areac - inter + 1e-9)
                miou = jnp.maximum(miou, iou)
            supp = jnp.max(miou) > _IOU_THR

            @pl.when(jnp.logical_not(supp))
            def _():
                vst1(kx1, nk, x1c)
                vst1(ky1, nk, y1c)
                vst1(kx2, nk, x2c)
                vst1(ky2, nk, y2c)
                vst1(karea, nk, areac)
                off = labc.astype(f32) * (_INPUT_SIZE + 1.0)
                vst1(outs, nk, jnp.broadcast_to(gmax, (16,)))
                vst1(outl, nk, labc)
                vst1(o1, nk, x1c - off)
                vst1(o2, nk, y1c - off)
                vst1(o3, nk, x2c - off)
                vst1(o4, nk, y2c - off)

            nk = nk + jnp.where(supp, jnp.int32(0), jnp.int32(1))

            vst1(sv, j, neg16)
            ci = j // 16
            q = ci // 16
            m1 = jnp.max(sv[pl.ds(pl.multiple_of(ci * 16, 16), 16)])
            vst1(cachev, ci, jnp.broadcast_to(m1, (16,)))
            m2 = jnp.max(cachev[pl.ds(pl.multiple_of(q * 16, 16), 16)])
            vst1(l2v, q, jnp.broadcast_to(m2, (16,)))
            gmax2, j2 = locate()
            return nk, gmax2, j2

        g0, j0 = locate()
        lax.while_loop(cond, body, (jnp.int32(0), g0, j0))

        pltpu.sync_copy(outs, south.at[b])
        pltpu.sync_copy(outl, louth.at[b])
        pltpu.sync_copy(o1, ox1h.at[b])
        pltpu.sync_copy(o2, oy1h.at[b])
        pltpu.sync_copy(o3, ox2h.at[b])
        pltpu.sync_copy(o4, oy2h.at[b])


def kernel(pred0, pred1, pred2):
    p0 = pred0.reshape(_N, 3 * 13 * 13, 85)
    p1 = pred1.reshape(_N, 3 * 26 * 26, 85)
    p2 = pred2.reshape(_N, 3 * 52 * 52, 85)

    def box_planes(p, seg):
        _, rows, n = seg
        q = jnp.transpose(p[:, :, :5], (0, 2, 1))
        q = jnp.pad(q, ((0, 0), (0, 0), (0, rows * 128 - n)))
        return q.reshape(_N, 5 * rows, 128)

    q0 = box_planes(p0, _SEGS[0])
    q1 = box_planes(p1, _SEGS[1])
    q2 = box_planes(p2, _SEGS[2])

    consts = (jnp.asarray(_CX), jnp.asarray(_CY), jnp.asarray(_AW),
              jnp.asarray(_AH), jnp.asarray(_VALID))

    f32 = jnp.float32
    dec_out = [jax.ShapeDtypeStruct((_N, _ROWS_TOT, 128), f32)] * 5
    dec_out.append(jax.ShapeDtypeStruct((_N, _ROWS_TOT, 128), jnp.int32))

    in_specs = [
        pl.BlockSpec((1, p0.shape[1], 85), lambda b: (b, 0, 0)),
        pl.BlockSpec((1, p1.shape[1], 85), lambda b: (b, 0, 0)),
        pl.BlockSpec((1, p2.shape[1], 85), lambda b: (b, 0, 0)),
        pl.BlockSpec((1, q0.shape[1], 128), lambda b: (b, 0, 0)),
        pl.BlockSpec((1, q1.shape[1], 128), lambda b: (b, 0, 0)),
        pl.BlockSpec((1, q2.shape[1], 128), lambda b: (b, 0, 0)),
    ] + [pl.BlockSpec((_ROWS_TOT, 128), lambda b: (0, 0))] * 5
    out_specs = [pl.BlockSpec((1, _ROWS_TOT, 128), lambda b: (b, 0, 0))] * 6

    x1o, y1o, x2o, y2o, sthr, lab = pl.pallas_call(
        _decode_body,
        grid=(_N,),
        in_specs=in_specs,
        out_specs=out_specs,
        out_shape=dec_out,
    )(p0, p1, p2, q0, q1, q2, *consts)

    flat = lambda a: a.reshape(_N, _NPAD)

    sc_nms = functools.partial(
        pl.kernel,
        out_type=[jax.ShapeDtypeStruct((_N, 128), f32),
                  jax.ShapeDtypeStruct((_N, 128), jnp.int32)]
        + [jax.ShapeDtypeStruct((_N, 128), f32)] * 4,
        mesh=plsc.VectorSubcoreMesh(core_axis_name="c", subcore_axis_name="s",
                                    num_cores=_NC, num_subcores=_NS),
        compiler_params=pltpu.CompilerParams(needs_layout_passes=False),
        scratch_types=[pltpu.VMEM((_NPAD,), f32)] * 5
        + [pltpu.VMEM((_NPAD,), jnp.int32),
           pltpu.VMEM((768,), f32),
           pltpu.VMEM((_L2PAD,), f32)]
        + [pltpu.VMEM((112,), f32)] * 5
        + [pltpu.VMEM((128,), f32),
           pltpu.VMEM((128,), jnp.int32)]
        + [pltpu.VMEM((128,), f32)] * 4,
    )(_sc_nms_body)

    sacc, lacc, x1a, y1a, x2a, y2a = sc_nms(
        flat(x1o), flat(y1o), flat(x2o), flat(y2o), flat(sthr), flat(lab))

    b_out = jnp.stack([x1a, y1a, x2a, y2a], axis=-1)[:, :_MAX_DET, :]
    s_out = sacc[:, :_MAX_DET]
    l_out = lacc[:, :_MAX_DET]
    return b_out, s_out, l_out


# speedup vs baseline: 17.2998x; 1.0304x over previous
import functools

import numpy as np
import jax
import jax.numpy as jnp
from jax import lax
from jax.experimental import pallas as pl
from jax.experimental.pallas import tpu as pltpu
from jax.experimental.pallas import tpu_sc as plsc

_INPUT_SIZE = 416.0
_ANCHORS = np.array([[[116, 90], [156, 198], [373, 326]],
                     [[30, 61], [62, 45], [59, 119]],
                     [[10, 13], [16, 30], [33, 23]]], dtype=np.float32)
_SCORE_THR = 0.5
_IOU_THR = 0.5
_MAX_DET = 100

_N = 16
_SCALES = (13, 26, 52)
_SEGS = []
_r0 = 0
for _S in _SCALES:
    _n = 3 * _S * _S
    _rows = -(-(-(-_n // 128)) // 8) * 8
    _SEGS.append((_r0, _rows, _n))
    _r0 += _rows
_ROWS_TOT = _r0
_NPAD = _ROWS_TOT * 128


def _make_consts():
    cx = np.zeros((_ROWS_TOT, 128), np.float32)
    cy = np.zeros((_ROWS_TOT, 128), np.float32)
    aw = np.ones((_ROWS_TOT, 128), np.float32)
    ah = np.ones((_ROWS_TOT, 128), np.float32)
    valid = np.zeros((_ROWS_TOT, 128), np.int32)
    for si, S in enumerate(_SCALES):
        r0, rows, n = _SEGS[si]
        idx = np.arange(rows * 128)
        a = np.minimum(idx // (S * S), 2)
        rem = idx % (S * S)
        cyv = (rem // S).astype(np.float32)
        cxv = (rem % S).astype(np.float32)
        blk = slice(r0, r0 + rows)
        cx[blk] = cxv.reshape(rows, 128)
        cy[blk] = cyv.reshape(rows, 128)
        aw[blk] = _ANCHORS[si][a, 0].reshape(rows, 128)
        ah[blk] = _ANCHORS[si][a, 1].reshape(rows, 128)
        v = (idx < n).astype(np.int32)
        valid[blk] = v.reshape(rows, 128)
    return cx, cy, aw, ah, valid


_CX, _CY, _AW, _AH, _VALID = _make_consts()


def _decode_body(p0, p1, p2, q0, q1, q2, cx, cy, aw, ah, valid,
                 x1o, y1o, x2o, y2o, sthr, lab):
    preds = (p0, p1, p2)
    planes = (q0, q1, q2)
    for si, S in enumerate(_SCALES):
        r0, rows, n = _SEGS[si]
        r = preds[si][0]
        lbl = jnp.argmax(r[:, 5:], axis=-1).astype(jnp.int32)

        pad = rows * 128 - n

        def to2d(v, fill):
            v = jnp.concatenate([v, jnp.full((pad,), fill, v.dtype)])
            return v.reshape(rows, 128)

        q = planes[si][0]
        s2 = q[0 * rows:1 * rows, :]
        tx = q[1 * rows:2 * rows, :]
        ty = q[2 * rows:3 * rows, :]
        tw = q[3 * rows:4 * rows, :]
        th = q[4 * rows:5 * rows, :]
        lb = to2d(lbl, 0)

        sl = slice(r0, r0 + rows)
        cxs, cys = cx[sl], cy[sl]
        aws, ahs = aw[sl], ah[sl]
        vals = valid[sl]

        stride = _INPUT_SIZE / S
        score = jax.nn.sigmoid(s2)
        bx = (jax.nn.sigmoid(tx) + cxs) * stride
        by = (jax.nn.sigmoid(ty) + cys) * stride
        bw = aws * jnp.exp(tw)
        bh = ahs * jnp.exp(th)
        x1 = jnp.clip(bx - bw / 2.0, 0.0, _INPUT_SIZE)
        y1 = jnp.clip(by - bh / 2.0, 0.0, _INPUT_SIZE)
        x2 = jnp.clip(bx + bw / 2.0, 0.0, _INPUT_SIZE)
        y2 = jnp.clip(by + bh / 2.0, 0.0, _INPUT_SIZE)

        off = lb.astype(jnp.float32) * (_INPUT_SIZE + 1.0)
        x1v = x1 + off
        y1v = y1 + off
        x2v = x2 + off
        y2v = y2 + off
        sv = jnp.where((vals != 0) & (score > _SCORE_THR), score,
                       -jnp.inf)

        x1o[0, sl, :] = x1v
        y1o[0, sl, :] = y1v
        x2o[0, sl, :] = x2v
        y2o[0, sl, :] = y2v
        sthr[0, sl, :] = sv
        lab[0, sl, :] = lb


_NC, _NS = 2, 16
_NVREG = _NPAD // 16
_NCACHE1 = -(-_NVREG // 16)
_L2PAD = 48


def _sc_nms_body(x1h, y1h, x2h, y2h, sh, labh,
                 south, louth, ox1h, oy1h, ox2h, oy2h,
                 x1v, y1v, x2v, y2v, sv, labv,
                 cachev, l2v, kx1, ky1, kx2, ky2, karea,
                 outs, outl, o1, o2, o3, o4):
    wid = lax.axis_index("s") * _NC + lax.axis_index("c")

    @pl.when(wid < _N)
    def _():
        b = wid
        pltpu.sync_copy(x1h.at[b], x1v)
        pltpu.sync_copy(y1h.at[b], y1v)
        pltpu.sync_copy(x2h.at[b], x2v)
        pltpu.sync_copy(y2h.at[b], y2v)
        pltpu.sync_copy(sh.at[b], sv)
        pltpu.sync_copy(labh.at[b], labv)

        neg = jnp.float32(-jnp.inf)
        f32 = jnp.float32
        iota = lax.iota(jnp.int32, 16)
        big_i = jnp.int32(1 << 30)
        sent16 = jnp.full((16,), 1e9, f32)
        zero16 = jnp.zeros((16,), f32)
        neg16 = jnp.full((16,), -jnp.inf, f32)
        lane0 = iota == 0

        def vld1(ref, i):
            return plsc.load_gather(ref, [jnp.full((16,), i, jnp.int32)])

        def vst1(ref, i, val16):
            plsc.store_scatter(ref, [jnp.full((16,), i, jnp.int32)],
                               val16, mask=lane0)

        def g16(ref, ci):
            return ref[ci // 8,
                       pl.ds(pl.multiple_of((ci % 8) * 16, 16), 16)]

        def vld2(ref, j):
            return plsc.load_gather(
                ref, [jnp.full((16,), j // 128, jnp.int32),
                      jnp.full((16,), j % 128, jnp.int32)])

        def vst2(ref, j, val16):
            plsc.store_scatter(
                ref, [jnp.full((16,), j // 128, jnp.int32),
                      jnp.full((16,), j % 128, jnp.int32)],
                val16, mask=lane0)

        for t in range(7):
            kx1[pl.ds(t * 16, 16)] = sent16
            ky1[pl.ds(t * 16, 16)] = sent16
            kx2[pl.ds(t * 16, 16)] = sent16
            ky2[pl.ds(t * 16, 16)] = sent16
            karea[pl.ds(t * 16, 16)] = zero16
        for t in range(8):
            outs[pl.ds(t * 16, 16)] = zero16
            outl[pl.ds(t * 16, 16)] = jnp.full((16,), -1, jnp.int32)
            o1[pl.ds(t * 16, 16)] = zero16
            o2[pl.ds(t * 16, 16)] = zero16
            o3[pl.ds(t * 16, 16)] = zero16
            o4[pl.ds(t * 16, 16)] = zero16

        for t in range(_NCACHE1, 48):
            cachev[pl.ds(t * 16, 16)] = neg16

        def cb(ci, _):
            m = jnp.max(g16(sv, ci))
            vst1(cachev, ci, jnp.broadcast_to(m, (16,)))
            return 0
        lax.fori_loop(0, _NVREG, cb, 0, unroll=8)

        for t in range(3):
            l2v[pl.ds(t * 16, 16)] = neg16

        def l2b(q, _):
            m = jnp.max(cachev[pl.ds(pl.multiple_of(q * 16, 16), 16)])
            vst1(l2v, q, jnp.broadcast_to(m, (16,)))
            return 0
        lax.fori_loop(0, _NCACHE1, l2b, 0, unroll=8)

        def locate():
            m = jnp.maximum(l2v[pl.ds(0, 16)], l2v[pl.ds(16, 16)])
            m = jnp.maximum(m, l2v[pl.ds(32, 16)])
            gmax = jnp.max(m)
            qidx = big_i
            for t in range(3):
                v = l2v[pl.ds(t * 16, 16)]
                qidx = jnp.minimum(
                    qidx, jnp.min(jnp.where(v == gmax, iota + t * 16, big_i)))
            q = jnp.minimum(qidx, _NCACHE1 - 1)
            cvec = cachev[pl.ds(pl.multiple_of(q * 16, 16), 16)]
            lq = jnp.min(jnp.where(cvec == gmax, iota, big_i))
            ci = jnp.minimum(q * 16 + lq, _NVREG - 1)
            svec = g16(sv, ci)
            ll = jnp.minimum(
                jnp.min(jnp.where(svec == gmax, iota, big_i)), 15)
            return gmax, ci * 16 + ll

        def cond(carry):
            nk, gmax, j = carry
            return (nk < _MAX_DET) & (gmax > neg)

        def body(carry):
            nk, gmax, j = carry
            x1c = vld2(x1v, j)
            y1c = vld2(y1v, j)
            x2c = vld2(x2v, j)
            y2c = vld2(y2v, j)
            labc = vld2(labv, j)
            areac = (x2c - x1c) * (y2c - y1c)

            miou = jnp.zeros((16,), f32)
            for t in range(7):
                ka1 = kx1[pl.ds(t * 16, 16)]
                kb1 = ky1[pl.ds(t * 16, 16)]
                ka2 = kx2[pl.ds(t * 16, 16)]
                kb2 = ky2[pl.ds(t * 16, 16)]
                kar = karea[pl.ds(t * 16, 16)]
                xx1 = jnp.maximum(ka1, x1c)
                yy1 = jnp.maximum(kb1, y1c)
                xx2 = jnp.minimum(ka2, x2c)
                yy2 = jnp.minimum(kb2, y2c)
                inter = (jnp.clip(xx2 - xx1, 0.0) *
                         jnp.clip(yy2 - yy1, 0.0))
                iou = inter / (kar + areac - inter + 1e-9)
                miou = jnp.maximum(miou, iou)
            supp = jnp.max(miou) > _IOU_THR

            @pl.when(jnp.logical_not(supp))
            def _():
                vst1(kx1, nk, x1c)
                vst1(ky1, nk, y1c)
                vst1(kx2, nk, x2c)
                vst1(ky2, nk, y2c)
                vst1(karea, nk, areac)
                off = labc.astype(f32) * (_INPUT_SIZE + 1.0)
                vst1(outs, nk, jnp.broadcast_to(gmax, (16,)))
                vst1(outl, nk, labc)
                vst1(o1, nk, x1c - off)
                vst1(o2, nk, y1c - off)
                vst1(o3, nk, x2c - off)
                vst1(o4, nk, y2c - off)

            nk = nk + jnp.where(supp, jnp.int32(0), jnp.int32(1))

            vst2(sv, j, neg16)
            ci = j // 16
            q = ci // 16
            m1 = jnp.max(g16(sv, ci))
            vst1(cachev, ci, jnp.broadcast_to(m1, (16,)))
            m2 = jnp.max(cachev[pl.ds(pl.multiple_of(q * 16, 16), 16)])
            vst1(l2v, q, jnp.broadcast_to(m2, (16,)))
            gmax2, j2 = locate()
            return nk, gmax2, j2

        g0, j0 = locate()
        lax.while_loop(cond, body, (jnp.int32(0), g0, j0))

        pltpu.sync_copy(outs, south.at[b])
        pltpu.sync_copy(outl, louth.at[b])
        pltpu.sync_copy(o1, ox1h.at[b])
        pltpu.sync_copy(o2, oy1h.at[b])
        pltpu.sync_copy(o3, ox2h.at[b])
        pltpu.sync_copy(o4, oy2h.at[b])


def kernel(pred0, pred1, pred2):
    p0 = pred0.reshape(_N, 3 * 13 * 13, 85)
    p1 = pred1.reshape(_N, 3 * 26 * 26, 85)
    p2 = pred2.reshape(_N, 3 * 52 * 52, 85)

    def box_planes(p, seg):
        _, rows, n = seg
        q = jnp.transpose(p[:, :, :5], (0, 2, 1))
        q = jnp.pad(q, ((0, 0), (0, 0), (0, rows * 128 - n)))
        return q.reshape(_N, 5 * rows, 128)

    q0 = box_planes(p0, _SEGS[0])
    q1 = box_planes(p1, _SEGS[1])
    q2 = box_planes(p2, _SEGS[2])

    consts = (jnp.asarray(_CX), jnp.asarray(_CY), jnp.asarray(_AW),
              jnp.asarray(_AH), jnp.asarray(_VALID))

    f32 = jnp.float32
    dec_out = [jax.ShapeDtypeStruct((_N, _ROWS_TOT, 128), f32)] * 5
    dec_out.append(jax.ShapeDtypeStruct((_N, _ROWS_TOT, 128), jnp.int32))

    in_specs = [
        pl.BlockSpec((1, p0.shape[1], 85), lambda b: (b, 0, 0)),
        pl.BlockSpec((1, p1.shape[1], 85), lambda b: (b, 0, 0)),
        pl.BlockSpec((1, p2.shape[1], 85), lambda b: (b, 0, 0)),
        pl.BlockSpec((1, q0.shape[1], 128), lambda b: (b, 0, 0)),
        pl.BlockSpec((1, q1.shape[1], 128), lambda b: (b, 0, 0)),
        pl.BlockSpec((1, q2.shape[1], 128), lambda b: (b, 0, 0)),
    ] + [pl.BlockSpec((_ROWS_TOT, 128), lambda b: (0, 0))] * 5
    out_specs = [pl.BlockSpec((1, _ROWS_TOT, 128), lambda b: (b, 0, 0))] * 6

    x1o, y1o, x2o, y2o, sthr, lab = pl.pallas_call(
        _decode_body,
        grid=(_N,),
        in_specs=in_specs,
        out_specs=out_specs,
        out_shape=dec_out,
    )(p0, p1, p2, q0, q1, q2, *consts)

    sc_nms = functools.partial(
        pl.kernel,
        out_type=[jax.ShapeDtypeStruct((_N, 128), f32),
                  jax.ShapeDtypeStruct((_N, 128), jnp.int32)]
        + [jax.ShapeDtypeStruct((_N, 128), f32)] * 4,
        mesh=plsc.VectorSubcoreMesh(core_axis_name="c", subcore_axis_name="s",
                                    num_cores=_NC, num_subcores=_NS),
        compiler_params=pltpu.CompilerParams(needs_layout_passes=False),
        scratch_types=[pltpu.VMEM((_ROWS_TOT, 128), f32)] * 5
        + [pltpu.VMEM((_ROWS_TOT, 128), jnp.int32),
           pltpu.VMEM((768,), f32),
           pltpu.VMEM((_L2PAD,), f32)]
        + [pltpu.VMEM((112,), f32)] * 5
        + [pltpu.VMEM((128,), f32),
           pltpu.VMEM((128,), jnp.int32)]
        + [pltpu.VMEM((128,), f32)] * 4,
    )(_sc_nms_body)

    sacc, lacc, x1a, y1a, x2a, y2a = sc_nms(x1o, y1o, x2o, y2o, sthr, lab)

    b_out = jnp.stack([x1a, y1a, x2a, y2a], axis=-1)[:, :_MAX_DET, :]
    s_out = sacc[:, :_MAX_DET]
    l_out = lacc[:, :_MAX_DET]
    return b_out, s_out, l_out


# speedup vs baseline: 21.5662x; 1.2466x over previous
import functools

import numpy as np
import jax
import jax.numpy as jnp
from jax import lax
from jax.experimental import pallas as pl
from jax.experimental.pallas import tpu as pltpu
from jax.experimental.pallas import tpu_sc as plsc

_INPUT_SIZE = 416.0
_ANCHORS = np.array([[[116, 90], [156, 198], [373, 326]],
                     [[30, 61], [62, 45], [59, 119]],
                     [[10, 13], [16, 30], [33, 23]]], dtype=np.float32)
_SCORE_THR = 0.5
_IOU_THR = 0.5
_MAX_DET = 100

_N = 16
_SCALES = (13, 26, 52)
_SEGS = []
_r0 = 0
for _S in _SCALES:
    _n = 3 * _S * _S
    _rows = -(-(-(-_n // 128)) // 8) * 8
    _SEGS.append((_r0, _rows, _n))
    _r0 += _rows
_ROWS_TOT = _r0
_NPAD = _ROWS_TOT * 128


def _make_consts():
    cx = np.zeros((_ROWS_TOT, 128), np.float32)
    cy = np.zeros((_ROWS_TOT, 128), np.float32)
    aw = np.ones((_ROWS_TOT, 128), np.float32)
    ah = np.ones((_ROWS_TOT, 128), np.float32)
    valid = np.zeros((_ROWS_TOT, 128), np.int32)
    for si, S in enumerate(_SCALES):
        r0, rows, n = _SEGS[si]
        idx = np.arange(rows * 128)
        a = np.minimum(idx // (S * S), 2)
        rem = idx % (S * S)
        cyv = (rem // S).astype(np.float32)
        cxv = (rem % S).astype(np.float32)
        blk = slice(r0, r0 + rows)
        cx[blk] = cxv.reshape(rows, 128)
        cy[blk] = cyv.reshape(rows, 128)
        aw[blk] = _ANCHORS[si][a, 0].reshape(rows, 128)
        ah[blk] = _ANCHORS[si][a, 1].reshape(rows, 128)
        v = (idx < n).astype(np.int32)
        valid[blk] = v.reshape(rows, 128)
    return cx, cy, aw, ah, valid


_CX, _CY, _AW, _AH, _VALID = _make_consts()


def _decode_body(p0, p1, p2, q0, q1, q2, cx, cy, aw, ah, valid,
                 x1o, y1o, x2o, y2o, sthr, lab):
    preds = (p0, p1, p2)
    planes = (q0, q1, q2)
    for si, S in enumerate(_SCALES):
        r0, rows, n = _SEGS[si]
        r = preds[si][...]
        lbl = jnp.argmax(r[..., 5:], axis=-1)
        lbl = lbl.reshape(n).astype(jnp.int32)

        pad = rows * 128 - n

        def to2d(v, fill):
            v = jnp.concatenate([v, jnp.full((pad,), fill, v.dtype)])
            return v.reshape(rows, 128)

        q = planes[si][0]
        s2 = q[0 * rows:1 * rows, :]
        tx = q[1 * rows:2 * rows, :]
        ty = q[2 * rows:3 * rows, :]
        tw = q[3 * rows:4 * rows, :]
        th = q[4 * rows:5 * rows, :]
        lb = to2d(lbl, 0)

        sl = slice(r0, r0 + rows)
        cxs, cys = cx[sl], cy[sl]
        aws, ahs = aw[sl], ah[sl]
        vals = valid[sl]

        stride = _INPUT_SIZE / S
        score = jax.nn.sigmoid(s2)
        bx = (jax.nn.sigmoid(tx) + cxs) * stride
        by = (jax.nn.sigmoid(ty) + cys) * stride
        bw = aws * jnp.exp(tw)
        bh = ahs * jnp.exp(th)
        x1 = jnp.clip(bx - bw / 2.0, 0.0, _INPUT_SIZE)
        y1 = jnp.clip(by - bh / 2.0, 0.0, _INPUT_SIZE)
        x2 = jnp.clip(bx + bw / 2.0, 0.0, _INPUT_SIZE)
        y2 = jnp.clip(by + bh / 2.0, 0.0, _INPUT_SIZE)

        off = lb.astype(jnp.float32) * (_INPUT_SIZE + 1.0)
        x1v = x1 + off
        y1v = y1 + off
        x2v = x2 + off
        y2v = y2 + off
        sv = jnp.where((vals != 0) & (score > _SCORE_THR), score,
                       -jnp.inf)

        x1o[0, sl, :] = x1v
        y1o[0, sl, :] = y1v
        x2o[0, sl, :] = x2v
        y2o[0, sl, :] = y2v
        sthr[0, sl, :] = sv
        lab[0, sl, :] = lb


_NC, _NS = 2, 16
_NVREG = _NPAD // 16
_NCACHE1 = -(-_NVREG // 16)
_L2PAD = 48


def _sc_nms_body(x1h, y1h, x2h, y2h, sh, labh,
                 south, louth, ox1h, oy1h, ox2h, oy2h,
                 x1v, y1v, x2v, y2v, sv, labv,
                 cachev, l2v, kx1, ky1, kx2, ky2, karea,
                 outs, outl, o1, o2, o3, o4):
    wid = lax.axis_index("s") * _NC + lax.axis_index("c")

    @pl.when(wid < _N)
    def _():
        b = wid
        pltpu.sync_copy(x1h.at[b], x1v)
        pltpu.sync_copy(y1h.at[b], y1v)
        pltpu.sync_copy(x2h.at[b], x2v)
        pltpu.sync_copy(y2h.at[b], y2v)
        pltpu.sync_copy(sh.at[b], sv)
        pltpu.sync_copy(labh.at[b], labv)

        neg = jnp.float32(-jnp.inf)
        f32 = jnp.float32
        iota = lax.iota(jnp.int32, 16)
        big_i = jnp.int32(1 << 30)
        sent16 = jnp.full((16,), 1e9, f32)
        zero16 = jnp.zeros((16,), f32)
        neg16 = jnp.full((16,), -jnp.inf, f32)
        lane0 = iota == 0

        def vld1(ref, i):
            return plsc.load_gather(ref, [jnp.full((16,), i, jnp.int32)])

        def vst1(ref, i, val16):
            plsc.store_scatter(ref, [jnp.full((16,), i, jnp.int32)],
                               val16, mask=lane0)

        def g16(ref, ci):
            return ref[ci // 8,
                       pl.ds(pl.multiple_of((ci % 8) * 16, 16), 16)]

        def vld2(ref, j):
            return plsc.load_gather(
                ref, [jnp.full((16,), j // 128, jnp.int32),
                      jnp.full((16,), j % 128, jnp.int32)])

        def vst2(ref, j, val16):
            plsc.store_scatter(
                ref, [jnp.full((16,), j // 128, jnp.int32),
                      jnp.full((16,), j % 128, jnp.int32)],
                val16, mask=lane0)

        for t in range(7):
            kx1[pl.ds(t * 16, 16)] = sent16
            ky1[pl.ds(t * 16, 16)] = sent16
            kx2[pl.ds(t * 16, 16)] = sent16
            ky2[pl.ds(t * 16, 16)] = sent16
            karea[pl.ds(t * 16, 16)] = zero16
        for t in range(8):
            outs[pl.ds(t * 16, 16)] = zero16
            outl[pl.ds(t * 16, 16)] = jnp.full((16,), -1, jnp.int32)
            o1[pl.ds(t * 16, 16)] = zero16
            o2[pl.ds(t * 16, 16)] = zero16
            o3[pl.ds(t * 16, 16)] = zero16
            o4[pl.ds(t * 16, 16)] = zero16

        for t in range(_NCACHE1, 48):
            cachev[pl.ds(t * 16, 16)] = neg16

        def cb(ci, _):
            m = jnp.max(g16(sv, ci))
            vst1(cachev, ci, jnp.broadcast_to(m, (16,)))
            return 0
        lax.fori_loop(0, _NVREG, cb, 0, unroll=8)

        for t in range(3):
            l2v[pl.ds(t * 16, 16)] = neg16

        def l2b(q, _):
            m = jnp.max(cachev[pl.ds(pl.multiple_of(q * 16, 16), 16)])
            vst1(l2v, q, jnp.broadcast_to(m, (16,)))
            return 0
        lax.fori_loop(0, _NCACHE1, l2b, 0, unroll=8)

        def locate():
            m = jnp.maximum(l2v[pl.ds(0, 16)], l2v[pl.ds(16, 16)])
            m = jnp.maximum(m, l2v[pl.ds(32, 16)])
            gmax = jnp.max(m)
            qidx = big_i
            for t in range(3):
                v = l2v[pl.ds(t * 16, 16)]
                qidx = jnp.minimum(
                    qidx, jnp.min(jnp.where(v == gmax, iota + t * 16, big_i)))
            q = jnp.minimum(qidx, _NCACHE1 - 1)
            cvec = cachev[pl.ds(pl.multiple_of(q * 16, 16), 16)]
            lq = jnp.min(jnp.where(cvec == gmax, iota, big_i))
            ci = jnp.minimum(q * 16 + lq, _NVREG - 1)
            svec = g16(sv, ci)
            ll = jnp.minimum(
                jnp.min(jnp.where(svec == gmax, iota, big_i)), 15)
            return gmax, ci * 16 + ll

        def cond(carry):
            nk, gmax, j = carry
            return (nk < _MAX_DET) & (gmax > neg)

        def body(carry):
            nk, gmax, j = carry
            x1c = vld2(x1v, j)
            y1c = vld2(y1v, j)
            x2c = vld2(x2v, j)
            y2c = vld2(y2v, j)
            labc = vld2(labv, j)
            areac = (x2c - x1c) * (y2c - y1c)

            miou = jnp.zeros((16,), f32)
            for t in range(7):
                ka1 = kx1[pl.ds(t * 16, 16)]
                kb1 = ky1[pl.ds(t * 16, 16)]
                ka2 = kx2[pl.ds(t * 16, 16)]
                kb2 = ky2[pl.ds(t * 16, 16)]
                kar = karea[pl.ds(t * 16, 16)]
                xx1 = jnp.maximum(ka1, x1c)
                yy1 = jnp.maximum(kb1, y1c)
                xx2 = jnp.minimum(ka2, x2c)
                yy2 = jnp.minimum(kb2, y2c)
                inter = (jnp.clip(xx2 - xx1, 0.0) *
                         jnp.clip(yy2 - yy1, 0.0))
                iou = inter / (kar + areac - inter + 1e-9)
                miou = jnp.maximum(miou, iou)
            supp = jnp.max(miou) > _IOU_THR

            @pl.when(jnp.logical_not(supp))
            def _():
                vst1(kx1, nk, x1c)
                vst1(ky1, nk, y1c)
                vst1(kx2, nk, x2c)
                vst1(ky2, nk, y2c)
                vst1(karea, nk, areac)
                off = labc.astype(f32) * (_INPUT_SIZE + 1.0)
                vst1(outs, nk, jnp.broadcast_to(gmax, (16,)))
                vst1(outl, nk, labc)
                vst1(o1, nk, x1c - off)
                vst1(o2, nk, y1c - off)
                vst1(o3, nk, x2c - off)
                vst1(o4, nk, y2c - off)

            nk = nk + jnp.where(supp, jnp.int32(0), jnp.int32(1))

            vst2(sv, j, neg16)
            ci = j // 16
            q = ci // 16
            m1 = jnp.max(g16(sv, ci))
            vst1(cachev, ci, jnp.broadcast_to(m1, (16,)))
            m2 = jnp.max(cachev[pl.ds(pl.multiple_of(q * 16, 16), 16)])
            vst1(l2v, q, jnp.broadcast_to(m2, (16,)))
            gmax2, j2 = locate()
            return nk, gmax2, j2

        g0, j0 = locate()
        lax.while_loop(cond, body, (jnp.int32(0), g0, j0))

        pltpu.sync_copy(outs, south.at[b])
        pltpu.sync_copy(outl, louth.at[b])
        pltpu.sync_copy(o1, ox1h.at[b])
        pltpu.sync_copy(o2, oy1h.at[b])
        pltpu.sync_copy(o3, ox2h.at[b])
        pltpu.sync_copy(o4, oy2h.at[b])


def kernel(pred0, pred1, pred2):
    p0 = pred0.reshape(_N * 3, 13, 13, 85)
    p1 = pred1.reshape(_N * 3, 26, 26, 85)
    p2 = pred2.reshape(_N * 3, 52, 52, 85)

    def box_planes(p, seg):
        _, rows, n = seg
        q = jnp.transpose(p[..., :5], (0, 4, 1, 2, 3))
        q = q.reshape(_N, 5, n)
        q = jnp.pad(q, ((0, 0), (0, 0), (0, rows * 128 - n)))
        return q.reshape(_N, 5 * rows, 128)

    q0 = box_planes(pred0, _SEGS[0])
    q1 = box_planes(pred1, _SEGS[1])
    q2 = box_planes(pred2, _SEGS[2])

    consts = (jnp.asarray(_CX), jnp.asarray(_CY), jnp.asarray(_AW),
              jnp.asarray(_AH), jnp.asarray(_VALID))

    f32 = jnp.float32
    dec_out = [jax.ShapeDtypeStruct((_N, _ROWS_TOT, 128), f32)] * 5
    dec_out.append(jax.ShapeDtypeStruct((_N, _ROWS_TOT, 128), jnp.int32))

    in_specs = [
        pl.BlockSpec((3, 13, 13, 85), lambda b: (b, 0, 0, 0)),
        pl.BlockSpec((3, 26, 26, 85), lambda b: (b, 0, 0, 0)),
        pl.BlockSpec((3, 52, 52, 85), lambda b: (b, 0, 0, 0)),
        pl.BlockSpec((1, q0.shape[1], 128), lambda b: (b, 0, 0)),
        pl.BlockSpec((1, q1.shape[1], 128), lambda b: (b, 0, 0)),
        pl.BlockSpec((1, q2.shape[1], 128), lambda b: (b, 0, 0)),
    ] + [pl.BlockSpec((_ROWS_TOT, 128), lambda b: (0, 0))] * 5
    out_specs = [pl.BlockSpec((1, _ROWS_TOT, 128), lambda b: (b, 0, 0))] * 6

    x1o, y1o, x2o, y2o, sthr, lab = pl.pallas_call(
        _decode_body,
        grid=(_N,),
        in_specs=in_specs,
        out_specs=out_specs,
        out_shape=dec_out,
    )(p0, p1, p2, q0, q1, q2, *consts)

    sc_nms = functools.partial(
        pl.kernel,
        out_type=[jax.ShapeDtypeStruct((_N, 128), f32),
                  jax.ShapeDtypeStruct((_N, 128), jnp.int32)]
        + [jax.ShapeDtypeStruct((_N, 128), f32)] * 4,
        mesh=plsc.VectorSubcoreMesh(core_axis_name="c", subcore_axis_name="s",
                                    num_cores=_NC, num_subcores=_NS),
        compiler_params=pltpu.CompilerParams(needs_layout_passes=False),
        scratch_types=[pltpu.VMEM((_ROWS_TOT, 128), f32)] * 5
        + [pltpu.VMEM((_ROWS_TOT, 128), jnp.int32),
           pltpu.VMEM((768,), f32),
           pltpu.VMEM((_L2PAD,), f32)]
        + [pltpu.VMEM((112,), f32)] * 5
        + [pltpu.VMEM((128,), f32),
           pltpu.VMEM((128,), jnp.int32)]
        + [pltpu.VMEM((128,), f32)] * 4,
    )(_sc_nms_body)

    sacc, lacc, x1a, y1a, x2a, y2a = sc_nms(x1o, y1o, x2o, y2o, sthr, lab)

    b_out = jnp.stack([x1a, y1a, x2a, y2a], axis=-1)[:, :_MAX_DET, :]
    s_out = sacc[:, :_MAX_DET]
    l_out = lacc[:, :_MAX_DET]
    return b_out, s_out, l_out
